# baseline (speedup 1.0000x reference)
import functools

import jax
import jax.numpy as jnp
from jax import lax
from jax.experimental import pallas as pl
from jax.experimental.pallas import tpu as pltpu
from jax.experimental.pallas import tpu_sc as plsc

NC = 2
NS = 16
NW = NC * NS
L = 16
IW = 128


@functools.lru_cache(maxsize=None)
def _make(T, B, D):
    BW = B // NW
    KG = BW // IW
    DT = D // 8
    mesh = plsc.VectorSubcoreMesh(
        core_axis_name="c", subcore_axis_name="s",
        num_cores=NC, num_subcores=NS)

    scratch = [
        pltpu.VMEM((KG, IW), jnp.int32),
        pltpu.VMEM((KG, IW), jnp.int32),
        pltpu.VMEM((BW, D), jnp.float32),
        pltpu.VMEM((BW, D), jnp.float32),
        pltpu.VMEM((DT * KG * 8 * IW,), jnp.float32),
        pltpu.VMEM((DT * KG * 8 * IW,), jnp.float32),
        pltpu.SemaphoreType.DMA,
        pltpu.SemaphoreType.DMA,
        pltpu.SemaphoreType.DMA,
        pltpu.SemaphoreType.DMA,
    ]

    @functools.partial(
        pl.kernel,
        out_type=jax.ShapeDtypeStruct((T, DT, B // IW, 8 * IW), jnp.float32),
        mesh=mesh,
        scratch_types=scratch,
        compiler_params=pltpu.CompilerParams(
            use_tc_tiling_on_sc=False, needs_layout_passes=False),
    )
    def k(ids_hbm, table_hbm, y_hbm, idx_a, idx_b, rows_a, rows_b, tr_a,
          tr_b, sem_a, sem_b, sem_sa, sem_sb):
        wid = lax.axis_index("s") * NC + lax.axis_index("c")
        bt0 = wid * KG
        lane = lax.iota(jnp.int32, L)
        rots = [(lane + kk) % L + half
                for half in range(0, D, L) for kk in range(L)]

        def load_idx(t, idx_ref):
            pltpu.sync_copy(ids_hbm.at[t, pl.ds(bt0, KG)], idx_ref)

        def fire(idx_ref, rows_ref, sem):
            for j in range(KG):
                pltpu.async_copy(table_hbm.at[idx_ref.at[j]],
                                 rows_ref.at[pl.ds(j * IW, IW)], sem)

        def drain_gather(rows_ref, sem):
            pltpu.make_async_copy(table_hbm.at[pl.ds(0, BW)], rows_ref,
                                  sem).wait()

        def transpose(rows_ref, tr_ref):
            @plsc.parallel_loop(0, BW // L, 1, unroll=16)
            def tr_step(g):
                bvec = g * L + lane
                gpart = lax.bitwise_or(
                    lax.bitwise_and(bvec, -IW) * 8,
                    lax.bitwise_and(bvec, IW - 1))
                for rot in rots:
                    v = plsc.load_gather(rows_ref, [bvec, rot])
                    rpart = lax.bitwise_or(
                        lax.bitwise_and(rot, D - 8) * (KG * IW),
                        lax.bitwise_and(rot, 7) * IW)
                    plsc.store_scatter(
                        tr_ref, [lax.bitwise_or(rpart, gpart)], v)

        def store(t, tr_ref, sem):
            for dblk in range(DT):
                for bt in range(KG):
                    pltpu.async_copy(
                        tr_ref.at[pl.ds((dblk * KG + bt) * 8 * IW, 8 * IW)],
                        y_hbm.at[t, dblk, bt0 + bt], sem)

        def drain_store(t, tr_ref, sem):
            for dblk in range(DT):
                for bt in range(KG):
                    pltpu.make_async_copy(
                        tr_ref.at[pl.ds((dblk * KG + bt) * 8 * IW, 8 * IW)],
                        y_hbm.at[t, dblk, bt0 + bt], sem).wait()

        load_idx(0, idx_a)
        fire(idx_a, rows_a, sem_a)

        def step2(i, carry):
            t0 = 2 * i
            load_idx(t0 + 1, idx_b)
            fire(idx_b, rows_b, sem_b)
            drain_gather(rows_a, sem_a)

            @pl.when(i > 0)
            def _():
                drain_store(t0 - 2, tr_a, sem_sa)

            transpose(rows_a, tr_a)
            store(t0, tr_a, sem_sa)

            @pl.when(t0 + 2 < T)
            def _():
                load_idx(t0 + 2, idx_a)
                fire(idx_a, rows_a, sem_a)

            drain_gather(rows_b, sem_b)

            @pl.when(i > 0)
            def _():
                drain_store(t0 - 1, tr_b, sem_sb)

            transpose(rows_b, tr_b)
            store(t0 + 1, tr_b, sem_sb)
            return carry

        lax.fori_loop(0, T // 2, step2, 0)
        drain_store(T - 2, tr_a, sem_sa)
        drain_store(T - 1, tr_b, sem_sb)

    return k


def kernel(object_ids, table):
    S, T = object_ids.shape
    D = table.shape[1]
    ids_t = object_ids.T.astype(jnp.int32).reshape(T, S // IW, IW)
    y = _make(T, S, D)(ids_t, table)
    y5 = y.reshape(T, D // 8, S // IW, 8, IW)
    return jnp.transpose(y5, (2, 4, 0, 1, 3)).reshape(S, T, D)


# speedup vs baseline: 1.3067x; 1.3067x over previous
import functools

import jax
import jax.numpy as jnp
from jax import lax
from jax.experimental import pallas as pl
from jax.experimental.pallas import tpu as pltpu
from jax.experimental.pallas import tpu_sc as plsc

NC = 2
NS = 16
NW = NC * NS
L = 16
IW = 128


@functools.lru_cache(maxsize=None)
def _make(T, B, D):
    BW = B // NW
    KG = BW // IW
    DT = D // 8
    mesh = plsc.VectorSubcoreMesh(
        core_axis_name="c", subcore_axis_name="s",
        num_cores=NC, num_subcores=NS)

    scratch = [
        pltpu.VMEM((KG, IW), jnp.int32),
        pltpu.VMEM((KG, IW), jnp.int32),
        pltpu.VMEM((BW, D), jnp.float32),
        pltpu.VMEM((BW, D), jnp.float32),
        pltpu.VMEM((DT * KG * 8 * IW,), jnp.float32),
        pltpu.VMEM((DT * KG * 8 * IW,), jnp.float32),
        pltpu.SemaphoreType.DMA,
        pltpu.SemaphoreType.DMA,
        pltpu.SemaphoreType.DMA,
        pltpu.SemaphoreType.DMA,
    ]

    @functools.partial(
        pl.kernel,
        out_type=jax.ShapeDtypeStruct((T, DT, B // IW, 8 * IW), jnp.float32),
        mesh=mesh,
        scratch_types=scratch,
        compiler_params=pltpu.CompilerParams(
            use_tc_tiling_on_sc=False, needs_layout_passes=False),
    )
    def k(ids_hbm, table_hbm, y_hbm, idx_a, idx_b, rows_a, rows_b, tr_a,
          tr_b, sem_a, sem_b, sem_sa, sem_sb):
        wid = lax.axis_index("s") * NC + lax.axis_index("c")
        bt0 = wid * KG
        lane = lax.iota(jnp.int32, L)
        rots = [(lane + kk) % L + half
                for half in range(0, D, L) for kk in range(L)]

        def load_idx(t, idx_ref):
            pltpu.sync_copy(ids_hbm.at[t, pl.ds(bt0, KG)], idx_ref)

        def fire(idx_ref, rows_ref, sem):
            for j in range(KG):
                pltpu.async_copy(table_hbm.at[idx_ref.at[j]],
                                 rows_ref.at[pl.ds(j * IW, IW)], sem)

        def drain_gather(rows_ref, sem):
            pltpu.make_async_copy(table_hbm.at[pl.ds(0, BW)], rows_ref,
                                  sem).wait()

        def transpose(rows_ref, tr_ref):
            @plsc.parallel_loop(0, BW // L, 1, unroll=8)
            def tr_step(g):
                bvec = g * L + lane
                gpart = lax.bitwise_or(
                    lax.bitwise_and(bvec, -IW) * 8,
                    lax.bitwise_and(bvec, IW - 1))
                for rot in rots:
                    v = plsc.load_gather(rows_ref, [bvec, rot])
                    rpart = lax.bitwise_or(
                        lax.bitwise_and(rot, D - 8) * (KG * IW),
                        lax.bitwise_and(rot, 7) * IW)
                    plsc.store_scatter(
                        tr_ref, [lax.bitwise_or(rpart, gpart)], v)

        def store(t, tr_ref, sem):
            for dblk in range(DT):
                for bt in range(KG):
                    pltpu.async_copy(
                        tr_ref.at[pl.ds((dblk * KG + bt) * 8 * IW, 8 * IW)],
                        y_hbm.at[t, dblk, bt0 + bt], sem)

        def drain_store(t, tr_ref, sem):
            for dblk in range(DT):
                for bt in range(KG):
                    pltpu.make_async_copy(
                        tr_ref.at[pl.ds((dblk * KG + bt) * 8 * IW, 8 * IW)],
                        y_hbm.at[t, dblk, bt0 + bt], sem).wait()

        load_idx(0, idx_a)
        fire(idx_a, rows_a, sem_a)

        def step2(i, carry):
            t0 = 2 * i
            load_idx(t0 + 1, idx_b)
            fire(idx_b, rows_b, sem_b)
            drain_gather(rows_a, sem_a)

            @pl.when(i > 0)
            def _():
                drain_store(t0 - 2, tr_a, sem_sa)

            transpose(rows_a, tr_a)
            store(t0, tr_a, sem_sa)

            @pl.when(t0 + 2 < T)
            def _():
                load_idx(t0 + 2, idx_a)
                fire(idx_a, rows_a, sem_a)

            drain_gather(rows_b, sem_b)

            @pl.when(i > 0)
            def _():
                drain_store(t0 - 1, tr_b, sem_sb)

            transpose(rows_b, tr_b)
            store(t0 + 1, tr_b, sem_sb)
            return carry

        lax.fori_loop(0, T // 2, step2, 0)
        drain_store(T - 2, tr_a, sem_sa)
        drain_store(T - 1, tr_b, sem_sb)

    return k


def kernel(object_ids, table):
    S, T = object_ids.shape
    D = table.shape[1]
    ids_t = object_ids.T.astype(jnp.int32).reshape(T, S // IW, IW)
    y = _make(T, S, D)(ids_t, table)
    y5 = y.reshape(T, D // 8, S // IW, 8, IW)
    return jnp.transpose(y5, (2, 4, 0, 1, 3)).reshape(S, T, D)
